# baseline (speedup 1.0000x reference)
import jax
import jax.numpy as jnp
from jax.experimental import pallas as pl
from jax.experimental.pallas import tpu as pltpu

_B, _N = 16, 32768
_TN = 2048
_NT = _N // _TN
_COUNT = float(_B * _N)
_EPS = 1e-5


def _mlp_kernel(xyz_ref, pts_ref,
                w0_ref, b0_ref, g0_ref, t0_ref,
                w1_ref, b1_ref, g1_ref, t1_ref,
                w2_ref, b2_ref, g2_ref, t2_ref,
                out_ref,
                s1, q1, s2, q2, s3, q3, smax, smin):
    p = pl.program_id(0)
    b = pl.program_id(1)
    nt = pl.program_id(2)

    @pl.when((p == 0) & (b == 0) & (nt == 0))
    def _init():
        s1[...] = jnp.zeros_like(s1)
        q1[...] = jnp.zeros_like(q1)
        s2[...] = jnp.zeros_like(s2)
        q2[...] = jnp.zeros_like(q2)
        s3[...] = jnp.zeros_like(s3)
        q3[...] = jnp.zeros_like(q3)
        smax[...] = jnp.full(smax.shape, -jnp.inf, smax.dtype)
        smin[...] = jnp.full(smin.shape, jnp.inf, smin.dtype)

    x = jnp.concatenate([xyz_ref[0], pts_ref[0]], axis=0)
    h1p = jnp.dot(w0_ref[...], x, preferred_element_type=jnp.float32) + b0_ref[...]

    @pl.when(p == 0)
    def _pass0():
        s1[...] += jnp.sum(h1p, axis=1, keepdims=True)
        q1[...] += jnp.sum(h1p * h1p, axis=1, keepdims=True)

    @pl.when(p >= 1)
    def _pass12():
        m1 = s1[...] / _COUNT
        v1 = q1[...] / _COUNT - m1 * m1
        a1 = g0_ref[...] * jax.lax.rsqrt(v1 + _EPS)
        c1 = t0_ref[...] - m1 * a1
        h1 = jnp.maximum(h1p * a1 + c1, 0.0)
        h2p = jnp.dot(w1_ref[...], h1, preferred_element_type=jnp.float32) + b1_ref[...]

        @pl.when(p == 1)
        def _pass1():
            s2[...] += jnp.sum(h2p, axis=1, keepdims=True)
            q2[...] += jnp.sum(h2p * h2p, axis=1, keepdims=True)

        @pl.when(p == 2)
        def _pass2():
            m2 = s2[...] / _COUNT
            v2 = q2[...] / _COUNT - m2 * m2
            a2 = g1_ref[...] * jax.lax.rsqrt(v2 + _EPS)
            c2 = t1_ref[...] - m2 * a2
            h2 = jnp.maximum(h2p * a2 + c2, 0.0)
            h3p = jnp.dot(w2_ref[...], h2, preferred_element_type=jnp.float32) + b2_ref[...]
            s3[...] += jnp.sum(h3p, axis=1, keepdims=True)
            q3[...] += jnp.sum(h3p * h3p, axis=1, keepdims=True)
            tile_max = jnp.max(h3p, axis=1, keepdims=True)
            tile_min = jnp.min(h3p, axis=1, keepdims=True)
            smax[b] = jnp.maximum(smax[b], tile_max)
            smin[b] = jnp.minimum(smin[b], tile_min)

    @pl.when((p == 2) & (b == _B - 1) & (nt == _NT - 1))
    def _finalize():
        m3 = s3[...] / _COUNT
        v3 = q3[...] / _COUNT - m3 * m3
        a3 = g2_ref[...] * jax.lax.rsqrt(v3 + _EPS)
        c3 = t2_ref[...] - m3 * a3
        picked = jnp.where(a3[None] >= 0.0, smax[...], smin[...])
        out_ref[...] = jnp.maximum(picked * a3[None] + c3[None], 0.0)


def kernel(xyz, points, W0, b0, g0, beta0, W1, b1, g1, beta1, W2, b2, g2, beta2):
    col = lambda v: v.reshape(-1, 1)
    wspec = lambda r, c: pl.BlockSpec((r, c), lambda p, b, nt: (0, 0))
    vspec = lambda r: pl.BlockSpec((r, 1), lambda p, b, nt: (0, 0))

    new_points = pl.pallas_call(
        _mlp_kernel,
        grid=(3, _B, _NT),
        in_specs=[
            pl.BlockSpec((1, 3, _TN), lambda p, b, nt: (b, 0, nt)),
            pl.BlockSpec((1, 29, _TN), lambda p, b, nt: (b, 0, nt)),
            wspec(32, 32), vspec(32), vspec(32), vspec(32),
            wspec(32, 32), vspec(32), vspec(32), vspec(32),
            wspec(64, 32), vspec(64), vspec(64), vspec(64),
        ],
        out_specs=pl.BlockSpec((_B, 64, 1), lambda p, b, nt: (0, 0, 0)),
        out_shape=jax.ShapeDtypeStruct((_B, 64, 1), jnp.float32),
        scratch_shapes=[
            pltpu.VMEM((32, 1), jnp.float32),
            pltpu.VMEM((32, 1), jnp.float32),
            pltpu.VMEM((32, 1), jnp.float32),
            pltpu.VMEM((32, 1), jnp.float32),
            pltpu.VMEM((64, 1), jnp.float32),
            pltpu.VMEM((64, 1), jnp.float32),
            pltpu.VMEM((_B, 64, 1), jnp.float32),
            pltpu.VMEM((_B, 64, 1), jnp.float32),
        ],
        compiler_params=pltpu.CompilerParams(
            dimension_semantics=("arbitrary", "arbitrary", "arbitrary"),
        ),
    )(xyz, points,
      W0, col(b0), col(g0), col(beta0),
      W1, col(b1), col(g1), col(beta1),
      W2, col(b2), col(g2), col(beta2))

    new_xyz = jnp.zeros((_B, 3, 1), dtype=xyz.dtype)
    return new_xyz, new_points
